# baseline (speedup 1.0000x reference)
import functools

import jax
import jax.numpy as jnp
import numpy as np
from jax import lax
from jax.experimental import pallas as pl
from jax.experimental.pallas import tpu as pltpu
from jax.experimental.pallas import tpu_sc as plsc

N = 10000
E = 320000
D = 128
H = 4
C = 32
OUT = 32
G = 64

NPAD = 10240
RBLK = 1024
NBLK = NPAD // RBLK
HD = D // 2
ACCW = 72
CHUNK = 32
NBUF = 5
EPT = 20800
ETOT_PAD = EPT * 16
NCHUNK = EPT // CHUNK
NACC = 10000
ACC_PT = NACC // 16
NCOPY = ACC_PT // 16


def _tc_front(x_pad, W, Amat):
    def body(x_ref, w_ref, am_ref, h_ref, a_ref):
        i = pl.program_id(0)
        h = jnp.dot(x_ref[...], w_ref[...], preferred_element_type=jnp.float32)
        h_ref[0] = h[:, :HD]
        h_ref[1] = h[:, HD:]
        a8 = lax.dot_general(am_ref[...], h, (((0,), (1,)), ((), ())),
                             preferred_element_type=jnp.float32)
        sent = jnp.logical_and(i == NBLK - 1,
                               lax.broadcasted_iota(jnp.int32, (1, RBLK), 1)
                               == RBLK - 1)
        rows = ((0, 1, 4, 5), (2, 3, 6, 7))
        for cc in range(2):
            for j in range(4):
                a_ref[cc, j] = jnp.where(sent, -1e30, a8[rows[cc][j]:rows[cc][j] + 1, :])[0]

    return pl.pallas_call(
        body,
        grid=(NBLK,),
        in_specs=[
            pl.BlockSpec((RBLK, D), lambda i: (i, 0)),
            pl.BlockSpec((D, D), lambda i: (0, 0)),
            pl.BlockSpec((D, 8), lambda i: (0, 0)),
        ],
        out_specs=[
            pl.BlockSpec((2, RBLK, HD), lambda i: (0, i, 0)),
            pl.BlockSpec((2, 4, RBLK), lambda i: (0, 0, i)),
        ],
        out_shape=[
            jax.ShapeDtypeStruct((2, NPAD, HD), jnp.float32),
            jax.ShapeDtypeStruct((2, 4, NPAD), jnp.float32),
        ],
    )(x_pad, W, Amat)


def _sc_edges(aTr, srcdst, h2):
    mesh = plsc.VectorSubcoreMesh(core_axis_name="c", subcore_axis_name="s")

    @functools.partial(
        pl.kernel,
        out_type=jax.ShapeDtypeStruct((2, NPAD, ACCW), jnp.float32),
        mesh=mesh,
        scratch_types=[
            pltpu.VMEM((4 * NPAD,), jnp.float32),
            pltpu.VMEM((EPT + NBUF * CHUNK,), jnp.int32),
            pltpu.VMEM((NBUF, CHUNK, HD), jnp.float32),
            pltpu.VMEM((NBUF, CHUNK, ACCW), jnp.float32),
            [pltpu.VMEM((CHUNK,), jnp.int32) for _ in range(NBUF)],
            [pltpu.VMEM((CHUNK,), jnp.int32) for _ in range(NBUF)],
            pltpu.VMEM_SHARED((NACC, ACCW), jnp.float32),
            pltpu.SemaphoreType.DMA,
            pltpu.SemaphoreType.DMA,
        ],
        compiler_params=pltpu.CompilerParams(needs_layout_passes=False,
                                             use_tc_tiling_on_sc=False),
    )
    def body(aT_hbm, sd_hbm, h_hbm, outp_hbm,
             aT_v, sd_v, rg, rs, srcbs, dstbs, out_acc, sg, ss):
        c = lax.axis_index("c")
        s = lax.axis_index("s")
        lane = lax.iota(jnp.int32, 16)
        zero16 = jnp.zeros((16,), jnp.float32)
        zero16i = jnp.zeros((16,), jnp.int32)
        mask16 = jnp.full((16,), 0xFFFF, jnp.int32)
        for b in range(NBUF):
            for k in range(CHUNK):
                for j in range(HD // 16):
                    rs[b, k, pl.ds(j * 16, 16)] = zero16
                rs[b, k, pl.ds(ACCW - 16, 16)] = zero16
            for half in range(CHUNK // 16):
                srcbs[b][pl.ds(half * 16, 16)] = zero16i
                dstbs[b][pl.ds(half * 16, 16)] = zero16i
        base = s * ACC_PT
        z16 = rs.at[0].at[pl.ds(0, 16)]

        def zero_body(it, carry):
            pltpu.sync_copy(z16, out_acc.at[pl.ds(base + it * 16, 16)])
            return carry

        lax.fori_loop(0, NCOPY, zero_body, 0)
        pltpu.sync_copy(rs.at[0].at[pl.ds(0, 1)],
                        out_acc.at[pl.ds(base + NCOPY * 16, 1)])
        pltpu.sync_copy(aT_hbm.at[c], aT_v)
        e0 = s * EPT
        pltpu.sync_copy(sd_hbm.at[pl.ds(e0, EPT)], sd_v.at[pl.ds(0, EPT)])
        for q in range(NBUF * CHUNK // 16):
            sd_v[pl.ds(EPT + q * 16, 16)] = zero16i
        plsc.subcore_barrier()

        hv = h_hbm.at[c]

        for b in range(NBUF):
            pltpu.async_copy(rs.at[b], out_acc.at[dstbs[b]], ss, add=True)
        for b in range(NBUF):
            for half in range(CHUNK // 16):
                sv = sd_v[pl.ds(b * CHUNK + half * 16, 16)] & mask16
                srcbs[b][pl.ds(half * 16, 16)] = sv
            pltpu.async_copy(hv.at[srcbs[b]], rg.at[b], sg)

        def halfstep(ci, b):
            rgb = rg.at[b]
            rsb = rs.at[b]
            srcb = srcbs[b]
            dstb = dstbs[b]
            off = ci * CHUNK
            pltpu.make_async_copy(hv.at[srcb], rgb, sg).wait()
            pltpu.make_async_copy(rsb, out_acc.at[dstb], ss).wait()
            for half in range(CHUNK // 16):
                sd16 = sd_v[pl.ds(off + half * 16, 16)]
                src16 = sd16 & mask16
                dst16 = lax.shift_right_logical(sd16, 16)
                dstb[pl.ds(half * 16, 16)] = dst16
                sdn = sd_v[pl.ds(off + NBUF * CHUNK + half * 16, 16)]
                srcb[pl.ds(half * 16, 16)] = sdn & mask16
                for hh in range(2):
                    asv = plsc.load_gather(aT_v, [src16 + (hh * NPAD)])
                    adv = plsc.load_gather(aT_v, [dst16 + ((2 + hh) * NPAD)])
                    e = asv + adv
                    e = jnp.where(e >= 0, e, 0.2 * e)
                    ee = jnp.exp(e)
                    plsc.store_scatter(rsb, [lane + half * 16,
                                             jnp.full((16,), HD + hh, jnp.int32)], ee)
            for k in range(CHUNK):
                wv = rsb[k, pl.ds(ACCW - 16, 16)]
                w0 = wv[8]
                w1 = wv[9]
                ws = (w0, w0, w1, w1)
                for j in range(HD // 16):
                    rsb[k, pl.ds(j * 16, 16)] = rgb[k, pl.ds(j * 16, 16)] * ws[j]
            pltpu.async_copy(rsb, out_acc.at[dstb], ss, add=True)
            pltpu.async_copy(hv.at[srcb], rgb, sg)

        def chunk_body(cg, carry):
            for b in range(NBUF):
                halfstep(cg * NBUF + b, b)
            return carry

        lax.fori_loop(0, NCHUNK // NBUF, chunk_body, 0)

        for b in range(NBUF):
            pltpu.make_async_copy(hv.at[srcbs[b]], rg.at[b], sg).wait()
            pltpu.make_async_copy(rs.at[b], out_acc.at[dstbs[b]], ss).wait()

        plsc.subcore_barrier()

        def wout_body(it, carry):
            r0 = base + it * 16
            pltpu.sync_copy(out_acc.at[pl.ds(r0, 16)], z16)
            pltpu.sync_copy(z16, outp_hbm.at[c, pl.ds(r0, 16)])
            return carry

        lax.fori_loop(0, NCOPY, wout_body, 0)
        r1 = base + NCOPY * 16
        pltpu.sync_copy(out_acc.at[pl.ds(r1, 1)], rs.at[0].at[pl.ds(0, 1)])
        pltpu.sync_copy(rs.at[0].at[pl.ds(0, 1)], outp_hbm.at[c, pl.ds(r1, 1)])

    return body(aTr, srcdst, h2)


def _tc_epilogue(outp, batchcol, E0, E1, bias2d, gamma2d, beta2d, lin_W,
                 lin_b2d):
    def body(op_ref, bc_ref, e0_ref, e1_ref, b_ref, g_ref, be_ref, lw_ref,
             lb_ref, st_ref, pe_ref, o_ref):
        i = pl.program_id(0)

        @pl.when(i < NBLK)
        def _():
            msum = jnp.concatenate([op_ref[0, :, 0:HD], op_ref[1, :, 0:HD]],
                                   axis=1)
            denb = (jnp.dot(op_ref[0, :, HD:ACCW], e0_ref[...],
                            preferred_element_type=jnp.float32)
                    + jnp.dot(op_ref[1, :, HD:ACCW], e1_ref[...],
                              preferred_element_type=jnp.float32))
            outv = msum / (denb + 1e-16) + b_ref[...]
            x1 = jnp.maximum(outv, 0.0)
            rowid = i * RBLK + lax.broadcasted_iota(jnp.int32, (RBLK, D), 0)
            x1 = jnp.where(rowid < N, x1, 0.0)
            bo = (bc_ref[...] == lax.broadcasted_iota(jnp.int32, (RBLK, G), 1)
                  ).astype(jnp.float32)
            x1e = jnp.concatenate([x1, jnp.ones_like(x1)], axis=1)
            pe = lax.dot_general(bo, x1e, (((0,), (0,)), ((), ())),
                                 preferred_element_type=jnp.float32)
            s1 = jnp.sum(x1, axis=0, keepdims=True)
            s2 = jnp.sum(x1 * x1, axis=0, keepdims=True)
            st = jnp.concatenate([s1, s2, jnp.zeros((6, D), jnp.float32)],
                                 axis=0)

            @pl.when(i == 0)
            def _():
                st_ref[...] = jnp.zeros_like(st_ref)
                pe_ref[...] = jnp.zeros_like(pe_ref)

            st_ref[...] += st
            pe_ref[...] += pe

        @pl.when(i == NBLK)
        def _():
            mean = st_ref[0:1, :] / float(N)
            var = st_ref[1:2, :] / float(N) - mean * mean
            sc = g_ref[...] / jnp.sqrt(var + 1e-5)
            P1 = pe_ref[:, 0:D]
            cntb = pe_ref[:, D:2 * D]
            pooled = P1 * sc + cntb * (be_ref[...] - mean * sc)
            logits = jnp.dot(pooled, lw_ref[...],
                             preferred_element_type=jnp.float32)
            o_ref[...] = jax.nn.sigmoid(logits + lb_ref[...])

    cl = lambda i: (0, jnp.minimum(i, NBLK - 1), 0)
    cl2 = lambda i: (jnp.minimum(i, NBLK - 1), 0)
    return pl.pallas_call(
        body,
        grid=(NBLK + 1,),
        in_specs=[
            pl.BlockSpec((2, RBLK, ACCW), cl),
            pl.BlockSpec((RBLK, 1), cl2),
            pl.BlockSpec((8, D), lambda i: (0, 0)),
            pl.BlockSpec((8, D), lambda i: (0, 0)),
            pl.BlockSpec((1, D), lambda i: (0, 0)),
            pl.BlockSpec((1, D), lambda i: (0, 0)),
            pl.BlockSpec((1, D), lambda i: (0, 0)),
            pl.BlockSpec((D, OUT), lambda i: (0, 0)),
            pl.BlockSpec((1, OUT), lambda i: (0, 0)),
        ],
        out_specs=[
            pl.BlockSpec((8, D), lambda i: (0, 0)),
            pl.BlockSpec((G, 2 * D), lambda i: (0, 0)),
            pl.BlockSpec((G, OUT), lambda i: (0, 0)),
        ],
        out_shape=[
            jax.ShapeDtypeStruct((8, D), jnp.float32),
            jax.ShapeDtypeStruct((G, 2 * D), jnp.float32),
            jax.ShapeDtypeStruct((G, OUT), jnp.float32),
        ],
    )(outp, batchcol, E0, E1, bias2d, gamma2d, beta2d, lin_W, lin_b2d)


def kernel(x, edge_index, batch, W, att_src, att_dst, bias_gat, gamma, beta,
           lin_W, lin_b):
    f32 = jnp.float32
    x_pad = jnp.zeros((NPAD, D), f32).at[:N].set(x)

    eye = jnp.eye(H, dtype=f32)
    Asrc = (eye[:, None, :] * att_src[:, :, None]).reshape(D, H)
    Adst = (eye[:, None, :] * att_dst[:, :, None]).reshape(D, H)
    Amat = jnp.concatenate([Asrc, Adst], axis=1)

    h2, aTr = _tc_front(x_pad, W, Amat)

    loop = jnp.arange(N, dtype=jnp.int32)
    npad_e = ETOT_PAD - (E + N)
    src = jnp.concatenate([edge_index[0].astype(jnp.int32), loop,
                           jnp.full((npad_e,), NPAD - 1, jnp.int32)])
    dst = jnp.concatenate([edge_index[1].astype(jnp.int32), loop,
                           jnp.zeros((npad_e,), jnp.int32)])
    srcdst = src | (dst << 16)

    outp = _sc_edges(aTr.reshape(2, 4 * NPAD), srcdst, h2)

    batchcol = jnp.full((NPAD, 1), G, jnp.int32).at[:N, 0].set(
        batch.astype(jnp.int32))
    hot = (jnp.eye(H, dtype=f32)[:, :, None] * jnp.ones((1, 1, C), f32)).reshape(H, D)
    E0 = jnp.concatenate([hot[0:2], jnp.zeros((6, D), f32)], axis=0)
    E1 = jnp.concatenate([hot[2:4], jnp.zeros((6, D), f32)], axis=0)

    stats, pe, out = _tc_epilogue(outp, batchcol, E0, E1,
                                  bias_gat.reshape(1, D), gamma.reshape(1, D),
                                  beta.reshape(1, D), lin_W,
                                  lin_b.reshape(1, OUT))
    del stats, pe
    return out


# speedup vs baseline: 1.0896x; 1.0896x over previous
import functools

import jax
import jax.numpy as jnp
import numpy as np
from jax import lax
from jax.experimental import pallas as pl
from jax.experimental.pallas import tpu as pltpu
from jax.experimental.pallas import tpu_sc as plsc

N = 10000
E = 320000
D = 128
H = 4
C = 32
OUT = 32
G = 64

NPAD = 10240
RBLK = 1024
NBLK = NPAD // RBLK
HD = D // 2
ACCW = 72
CHUNK = 16
NBUF = 8
EPT = 20736
ETOT_PAD = EPT * 16
NCHUNK = EPT // CHUNK
NACC = 10000
ACC_PT = NACC // 16
NCOPY = ACC_PT // 16


def _tc_front(x_pad, W, Amat):
    def body(x_ref, w_ref, am_ref, h_ref, a_ref):
        i = pl.program_id(0)
        h = jnp.dot(x_ref[...], w_ref[...], preferred_element_type=jnp.float32)
        h_ref[0] = h[:, :HD]
        h_ref[1] = h[:, HD:]
        a8 = lax.dot_general(am_ref[...], h, (((0,), (1,)), ((), ())),
                             preferred_element_type=jnp.float32)
        sent = jnp.logical_and(i == NBLK - 1,
                               lax.broadcasted_iota(jnp.int32, (1, RBLK), 1)
                               == RBLK - 1)
        rows = ((0, 1, 4, 5), (2, 3, 6, 7))
        for cc in range(2):
            for j in range(4):
                a_ref[cc, j] = jnp.where(sent, -1e30, a8[rows[cc][j]:rows[cc][j] + 1, :])[0]

    return pl.pallas_call(
        body,
        grid=(NBLK,),
        in_specs=[
            pl.BlockSpec((RBLK, D), lambda i: (i, 0)),
            pl.BlockSpec((D, D), lambda i: (0, 0)),
            pl.BlockSpec((D, 8), lambda i: (0, 0)),
        ],
        out_specs=[
            pl.BlockSpec((2, RBLK, HD), lambda i: (0, i, 0)),
            pl.BlockSpec((2, 4, RBLK), lambda i: (0, 0, i)),
        ],
        out_shape=[
            jax.ShapeDtypeStruct((2, NPAD, HD), jnp.float32),
            jax.ShapeDtypeStruct((2, 4, NPAD), jnp.float32),
        ],
    )(x_pad, W, Amat)


def _sc_edges(aTr, srcdst, h2):
    mesh = plsc.VectorSubcoreMesh(core_axis_name="c", subcore_axis_name="s")

    @functools.partial(
        pl.kernel,
        out_type=jax.ShapeDtypeStruct((2, NPAD, ACCW), jnp.float32),
        mesh=mesh,
        scratch_types=[
            pltpu.VMEM((4 * NPAD,), jnp.float32),
            pltpu.VMEM((EPT + NBUF * CHUNK,), jnp.int32),
            pltpu.VMEM((NBUF, CHUNK, HD), jnp.float32),
            pltpu.VMEM((NBUF, CHUNK, ACCW), jnp.float32),
            pltpu.VMEM_SHARED((NACC, ACCW), jnp.float32),
            pltpu.SemaphoreType.DMA,
            pltpu.SemaphoreType.DMA,
        ],
        compiler_params=pltpu.CompilerParams(needs_layout_passes=False,
                                             use_tc_tiling_on_sc=False),
    )
    def body(aT_hbm, sd_hbm, h_hbm, outp_hbm,
             aT_v, sd_v, rg, rs, out_acc, sg, ss):
        c = lax.axis_index("c")
        s = lax.axis_index("s")
        lane = lax.iota(jnp.int32, 16)
        zero16 = jnp.zeros((16,), jnp.float32)
        zero16i = jnp.zeros((16,), jnp.int32)
        mask16 = jnp.full((16,), 0xFFFF, jnp.int32)
        for b in range(NBUF):
            for k in range(CHUNK):
                for j in range(HD // 16):
                    rs[b, k, pl.ds(j * 16, 16)] = zero16
                rs[b, k, pl.ds(ACCW - 16, 16)] = zero16
        base = s * ACC_PT

        def zero_body(it, carry):
            pltpu.sync_copy(rs.at[0], out_acc.at[pl.ds(base + it * 16, 16)])
            return carry

        lax.fori_loop(0, NCOPY, zero_body, 0)
        pltpu.sync_copy(rs.at[0].at[pl.ds(0, 1)],
                        out_acc.at[pl.ds(base + NCOPY * 16, 1)])
        pltpu.sync_copy(aT_hbm.at[c], aT_v)
        e0 = s * EPT
        pltpu.sync_copy(sd_hbm.at[pl.ds(e0, EPT)], sd_v.at[pl.ds(0, EPT)])
        for q in range(NBUF):
            sd_v[pl.ds(EPT + q * 16, 16)] = zero16i
        plsc.subcore_barrier()

        hv = h_hbm.at[c]
        s16p = sd_v[pl.ds(0, 16)] & mask16

        for b in range(NBUF):
            pltpu.async_copy(rs.at[b], out_acc.at[zero16i], ss, add=True)
        for b in range(NBUF):
            sb = sd_v[pl.ds(b * CHUNK, 16)] & mask16
            pltpu.async_copy(hv.at[sb], rg.at[b], sg)

        def halfstep(ci, b):
            rgb = rg.at[b]
            rsb = rs.at[b]
            off = ci * CHUNK
            sd16 = sd_v[pl.ds(off, 16)]
            src16 = sd16 & mask16
            dst16 = lax.shift_right_logical(sd16, 16)
            pltpu.make_async_copy(hv.at[src16], rgb, sg).wait()
            pltpu.make_async_copy(rsb, out_acc.at[dst16], ss).wait()
            ees = []
            for hh in range(2):
                asv = plsc.load_gather(aT_v, [src16 + (hh * NPAD)])
                adv = plsc.load_gather(aT_v, [dst16 + ((2 + hh) * NPAD)])
                e = asv + adv
                e = jnp.where(e >= 0, e, 0.2 * e)
                ee = jnp.exp(e)
                ees.append(ee)
                plsc.store_scatter(rsb, [lane, jnp.full((16,), HD + hh, jnp.int32)], ee)
            for k in range(CHUNK):
                kf = jnp.full((16,), k, jnp.int32)
                w0 = ees[0].at[kf].get(mode="promise_in_bounds")
                w1 = ees[1].at[kf].get(mode="promise_in_bounds")
                ws = (w0, w0, w1, w1)
                for j in range(HD // 16):
                    rsb[k, pl.ds(j * 16, 16)] = rgb[k, pl.ds(j * 16, 16)] * ws[j]
            pltpu.async_copy(rsb, out_acc.at[dst16], ss, add=True)
            srcn = sd_v[pl.ds(off + NBUF * CHUNK, 16)] & mask16
            pltpu.async_copy(hv.at[srcn], rgb, sg)

        def chunk_body(cg, carry):
            for b in range(NBUF):
                halfstep(cg * NBUF + b, b)
            return carry

        lax.fori_loop(0, NCHUNK // NBUF, chunk_body, 0)

        for b in range(NBUF):
            pltpu.make_async_copy(hv.at[s16p], rg.at[b], sg).wait()
            pltpu.make_async_copy(rs.at[b], out_acc.at[zero16i], ss).wait()

        plsc.subcore_barrier()

        def wout_body(it, carry):
            r0 = base + it * 16
            pltpu.sync_copy(out_acc.at[pl.ds(r0, 16)], rs.at[0])
            pltpu.sync_copy(rs.at[0], outp_hbm.at[c, pl.ds(r0, 16)])
            return carry

        lax.fori_loop(0, NCOPY, wout_body, 0)
        r1 = base + NCOPY * 16
        pltpu.sync_copy(out_acc.at[pl.ds(r1, 1)], rs.at[0].at[pl.ds(0, 1)])
        pltpu.sync_copy(rs.at[0].at[pl.ds(0, 1)], outp_hbm.at[c, pl.ds(r1, 1)])

    return body(aTr, srcdst, h2)


def _tc_epilogue(outp, batchcol, E0, E1, bias2d, gamma2d, beta2d, lin_W,
                 lin_b2d):
    def body(op_ref, bc_ref, e0_ref, e1_ref, b_ref, g_ref, be_ref, lw_ref,
             lb_ref, st_ref, pe_ref, o_ref):
        i = pl.program_id(0)

        @pl.when(i < NBLK)
        def _():
            msum = jnp.concatenate([op_ref[0, :, 0:HD], op_ref[1, :, 0:HD]],
                                   axis=1)
            denb = (jnp.dot(op_ref[0, :, HD:ACCW], e0_ref[...],
                            preferred_element_type=jnp.float32)
                    + jnp.dot(op_ref[1, :, HD:ACCW], e1_ref[...],
                              preferred_element_type=jnp.float32))
            outv = msum / (denb + 1e-16) + b_ref[...]
            x1 = jnp.maximum(outv, 0.0)
            rowid = i * RBLK + lax.broadcasted_iota(jnp.int32, (RBLK, D), 0)
            x1 = jnp.where(rowid < N, x1, 0.0)
            bo = (bc_ref[...] == lax.broadcasted_iota(jnp.int32, (RBLK, G), 1)
                  ).astype(jnp.float32)
            x1e = jnp.concatenate([x1, jnp.ones_like(x1)], axis=1)
            pe = lax.dot_general(bo, x1e, (((0,), (0,)), ((), ())),
                                 preferred_element_type=jnp.float32)
            s1 = jnp.sum(x1, axis=0, keepdims=True)
            s2 = jnp.sum(x1 * x1, axis=0, keepdims=True)
            st = jnp.concatenate([s1, s2, jnp.zeros((6, D), jnp.float32)],
                                 axis=0)

            @pl.when(i == 0)
            def _():
                st_ref[...] = jnp.zeros_like(st_ref)
                pe_ref[...] = jnp.zeros_like(pe_ref)

            st_ref[...] += st
            pe_ref[...] += pe

        @pl.when(i == NBLK)
        def _():
            mean = st_ref[0:1, :] / float(N)
            var = st_ref[1:2, :] / float(N) - mean * mean
            sc = g_ref[...] / jnp.sqrt(var + 1e-5)
            P1 = pe_ref[:, 0:D]
            cntb = pe_ref[:, D:2 * D]
            pooled = P1 * sc + cntb * (be_ref[...] - mean * sc)
            logits = jnp.dot(pooled, lw_ref[...],
                             preferred_element_type=jnp.float32)
            o_ref[...] = jax.nn.sigmoid(logits + lb_ref[...])

    cl = lambda i: (0, jnp.minimum(i, NBLK - 1), 0)
    cl2 = lambda i: (jnp.minimum(i, NBLK - 1), 0)
    return pl.pallas_call(
        body,
        grid=(NBLK + 1,),
        in_specs=[
            pl.BlockSpec((2, RBLK, ACCW), cl),
            pl.BlockSpec((RBLK, 1), cl2),
            pl.BlockSpec((8, D), lambda i: (0, 0)),
            pl.BlockSpec((8, D), lambda i: (0, 0)),
            pl.BlockSpec((1, D), lambda i: (0, 0)),
            pl.BlockSpec((1, D), lambda i: (0, 0)),
            pl.BlockSpec((1, D), lambda i: (0, 0)),
            pl.BlockSpec((D, OUT), lambda i: (0, 0)),
            pl.BlockSpec((1, OUT), lambda i: (0, 0)),
        ],
        out_specs=[
            pl.BlockSpec((8, D), lambda i: (0, 0)),
            pl.BlockSpec((G, 2 * D), lambda i: (0, 0)),
            pl.BlockSpec((G, OUT), lambda i: (0, 0)),
        ],
        out_shape=[
            jax.ShapeDtypeStruct((8, D), jnp.float32),
            jax.ShapeDtypeStruct((G, 2 * D), jnp.float32),
            jax.ShapeDtypeStruct((G, OUT), jnp.float32),
        ],
    )(outp, batchcol, E0, E1, bias2d, gamma2d, beta2d, lin_W, lin_b2d)


def kernel(x, edge_index, batch, W, att_src, att_dst, bias_gat, gamma, beta,
           lin_W, lin_b):
    f32 = jnp.float32
    x_pad = jnp.zeros((NPAD, D), f32).at[:N].set(x)

    eye = jnp.eye(H, dtype=f32)
    Asrc = (eye[:, None, :] * att_src[:, :, None]).reshape(D, H)
    Adst = (eye[:, None, :] * att_dst[:, :, None]).reshape(D, H)
    Amat = jnp.concatenate([Asrc, Adst], axis=1)

    h2, aTr = _tc_front(x_pad, W, Amat)

    loop = jnp.arange(N, dtype=jnp.int32)
    npad_e = ETOT_PAD - (E + N)
    src = jnp.concatenate([edge_index[0].astype(jnp.int32), loop,
                           jnp.full((npad_e,), NPAD - 1, jnp.int32)])
    dst = jnp.concatenate([edge_index[1].astype(jnp.int32), loop,
                           jnp.zeros((npad_e,), jnp.int32)])
    srcdst = src | (dst << 16)

    outp = _sc_edges(aTr.reshape(2, 4 * NPAD), srcdst, h2)

    batchcol = jnp.full((NPAD, 1), G, jnp.int32).at[:N, 0].set(
        batch.astype(jnp.int32))
    hot = (jnp.eye(H, dtype=f32)[:, :, None] * jnp.ones((1, 1, C), f32)).reshape(H, D)
    E0 = jnp.concatenate([hot[0:2], jnp.zeros((6, D), f32)], axis=0)
    E1 = jnp.concatenate([hot[2:4], jnp.zeros((6, D), f32)], axis=0)

    stats, pe, out = _tc_epilogue(outp, batchcol, E0, E1,
                                  bias_gat.reshape(1, D), gamma.reshape(1, D),
                                  beta.reshape(1, D), lin_W,
                                  lin_b.reshape(1, OUT))
    del stats, pe
    return out


# speedup vs baseline: 1.0898x; 1.0002x over previous
import functools

import jax
import jax.numpy as jnp
from jax import lax
from jax.experimental import pallas as pl
from jax.experimental.pallas import tpu as pltpu
from jax.experimental.pallas import tpu_sc as plsc

N = 10000
E = 320000
D = 128
H = 4
C = 32
OUT = 32
G = 64

NPAD = 10240
RBLK = 1024
NBLK = NPAD // RBLK
HD = D // 2
ACCW = 72
CHUNK = 16
NBUF = 8
EPT = 20736
ETOT_PAD = EPT * 16
NCHUNK = EPT // CHUNK
NACC = 10000
ACC_PT = NACC // 16
NCOPY = ACC_PT // 16


def _tc_front(x_pad, W, Amat):
    def body(x_ref, w_ref, am_ref, h_ref, a_ref):
        i = pl.program_id(0)
        h = jnp.dot(x_ref[...], w_ref[...], preferred_element_type=jnp.float32)
        h_ref[0] = h[:, :HD]
        h_ref[1] = h[:, HD:]
        a8 = lax.dot_general(am_ref[...], h, (((0,), (1,)), ((), ())),
                             preferred_element_type=jnp.float32)
        sent = jnp.logical_and(i == NBLK - 1,
                               lax.broadcasted_iota(jnp.int32, (1, RBLK), 1)
                               == RBLK - 1)
        rows = ((0, 1, 4, 5), (2, 3, 6, 7))
        for cc in range(2):
            for j in range(4):
                a_ref[cc, j] = jnp.where(sent, -1e30, a8[rows[cc][j]:rows[cc][j] + 1, :])[0]

    return pl.pallas_call(
        body,
        grid=(NBLK,),
        in_specs=[
            pl.BlockSpec((RBLK, D), lambda i: (i, 0)),
            pl.BlockSpec((D, D), lambda i: (0, 0)),
            pl.BlockSpec((D, 8), lambda i: (0, 0)),
        ],
        out_specs=[
            pl.BlockSpec((2, RBLK, HD), lambda i: (0, i, 0)),
            pl.BlockSpec((2, 4, RBLK), lambda i: (0, 0, i)),
        ],
        out_shape=[
            jax.ShapeDtypeStruct((2, NPAD, HD), jnp.float32),
            jax.ShapeDtypeStruct((2, 4, NPAD), jnp.float32),
        ],
    )(x_pad, W, Amat)


def _sc_edges(aTr, srcdst, h2):
    mesh = plsc.VectorSubcoreMesh(core_axis_name="c", subcore_axis_name="s")

    @functools.partial(
        pl.kernel,
        out_type=jax.ShapeDtypeStruct((2, NPAD, ACCW), jnp.float32),
        mesh=mesh,
        scratch_types=[
            pltpu.VMEM((4 * NPAD,), jnp.float32),
            pltpu.VMEM((EPT + NBUF * CHUNK,), jnp.int32),
            pltpu.VMEM((NBUF, CHUNK, HD), jnp.float32),
            pltpu.VMEM((NBUF, CHUNK, ACCW), jnp.float32),
            pltpu.VMEM_SHARED((NACC, ACCW), jnp.float32),
            pltpu.SemaphoreType.DMA,
            pltpu.SemaphoreType.DMA,
        ],
        compiler_params=pltpu.CompilerParams(needs_layout_passes=False,
                                             use_tc_tiling_on_sc=False),
    )
    def body(aT_hbm, sd_hbm, h_hbm, outp_hbm,
             aT_v, sd_v, rg, rs, out_acc, sg, ss):
        c = lax.axis_index("c")
        s = lax.axis_index("s")
        lane = lax.iota(jnp.int32, 16)
        zero16 = jnp.zeros((16,), jnp.float32)
        zero16i = jnp.zeros((16,), jnp.int32)
        mask16 = jnp.full((16,), 0xFFFF, jnp.int32)
        for b in range(NBUF):
            for k in range(CHUNK):
                for j in range(HD // 16):
                    rs[b, k, pl.ds(j * 16, 16)] = zero16
                rs[b, k, pl.ds(ACCW - 16, 16)] = zero16
        base = s * ACC_PT

        def zero_body(it, carry):
            pltpu.sync_copy(rs.at[0], out_acc.at[pl.ds(base + it * 16, 16)])
            return carry

        lax.fori_loop(0, NCOPY, zero_body, 0)
        pltpu.sync_copy(rs.at[0].at[pl.ds(0, 1)],
                        out_acc.at[pl.ds(base + NCOPY * 16, 1)])
        pltpu.sync_copy(aT_hbm.at[c], aT_v)
        e0 = s * EPT
        pltpu.sync_copy(sd_hbm.at[pl.ds(e0, EPT)], sd_v.at[pl.ds(0, EPT)])
        for q in range(NBUF):
            sd_v[pl.ds(EPT + q * 16, 16)] = zero16i
        plsc.subcore_barrier()

        hv = h_hbm.at[c]
        s16p = sd_v[pl.ds(0, 16)] & mask16

        for b in range(NBUF):
            pltpu.async_copy(rs.at[b], out_acc.at[zero16i], ss, add=True)
        for b in range(NBUF):
            sb = sd_v[pl.ds(b * CHUNK, 16)] & mask16
            pltpu.async_copy(hv.at[sb], rg.at[b], sg)

        def halfstep(ci, b):
            rgb = rg.at[b]
            rsb = rs.at[b]
            off = ci * CHUNK
            sd16 = sd_v[pl.ds(off, 16)]
            src16 = sd16 & mask16
            dst16 = lax.shift_right_logical(sd16, 16)
            pltpu.make_async_copy(hv.at[src16], rgb, sg).wait()
            pltpu.make_async_copy(rsb, out_acc.at[dst16], ss).wait()
            ees = []
            for hh in range(2):
                asv = plsc.load_gather(aT_v, [src16 + (hh * NPAD)])
                adv = plsc.load_gather(aT_v, [dst16 + ((2 + hh) * NPAD)])
                e = asv + adv
                e = jnp.where(e >= 0, e, 0.2 * e)
                ee = jnp.exp(e)
                ees.append(ee)
                plsc.store_scatter(rsb, [lane, jnp.full((16,), HD + hh, jnp.int32)], ee)
            for k in range(CHUNK):
                kf = jnp.full((16,), k, jnp.int32)
                w0 = ees[0].at[kf].get(mode="promise_in_bounds")
                w1 = ees[1].at[kf].get(mode="promise_in_bounds")
                ws = (w0, w0, w1, w1)
                for j in range(HD // 16):
                    rsb[k, pl.ds(j * 16, 16)] = rgb[k, pl.ds(j * 16, 16)] * ws[j]
            pltpu.async_copy(rsb, out_acc.at[dst16], ss, add=True)
            srcn = sd_v[pl.ds(off + NBUF * CHUNK, 16)] & mask16
            pltpu.async_copy(hv.at[srcn], rgb, sg)

        def chunk_body(cg, carry):
            for b in range(NBUF):
                halfstep(cg * NBUF + b, b)
            return carry

        lax.fori_loop(0, NCHUNK // NBUF, chunk_body, 0)

        for b in range(NBUF):
            pltpu.make_async_copy(hv.at[s16p], rg.at[b], sg).wait()
            pltpu.make_async_copy(rs.at[b], out_acc.at[zero16i], ss).wait()

        plsc.subcore_barrier()

        def wout_body(it, carry):
            r0 = base + it * 16
            pltpu.sync_copy(out_acc.at[pl.ds(r0, 16)], rs.at[0])
            pltpu.sync_copy(rs.at[0], outp_hbm.at[c, pl.ds(r0, 16)])
            return carry

        lax.fori_loop(0, NCOPY, wout_body, 0)
        r1 = base + NCOPY * 16
        pltpu.sync_copy(out_acc.at[pl.ds(r1, 1)], rs.at[0].at[pl.ds(0, 1)])
        pltpu.sync_copy(rs.at[0].at[pl.ds(0, 1)], outp_hbm.at[c, pl.ds(r1, 1)])

    return body(aTr, srcdst, h2)


def _tc_epilogue(outp, batchcol, E0, E1, bias2d, gamma2d, beta2d, lin_W,
                 lin_b2d):
    def body(op_ref, bc_ref, e0_ref, e1_ref, b_ref, g_ref, be_ref, lw_ref,
             lb_ref, st_ref, pe_ref, o_ref):
        i = pl.program_id(0)

        @pl.when(i < NBLK)
        def _():
            msum = jnp.concatenate([op_ref[0, :, 0:HD], op_ref[1, :, 0:HD]],
                                   axis=1)
            denb = (jnp.dot(op_ref[0, :, HD:ACCW], e0_ref[...],
                            preferred_element_type=jnp.float32)
                    + jnp.dot(op_ref[1, :, HD:ACCW], e1_ref[...],
                              preferred_element_type=jnp.float32))
            outv = msum / (denb + 1e-16) + b_ref[...]
            x1 = jnp.maximum(outv, 0.0)
            rowid = i * RBLK + lax.broadcasted_iota(jnp.int32, (RBLK, D), 0)
            x1 = jnp.where(rowid < N, x1, 0.0)
            bo = (bc_ref[...] == lax.broadcasted_iota(jnp.int32, (RBLK, G), 1)
                  ).astype(jnp.float32)
            x1e = jnp.concatenate([x1, jnp.ones_like(x1)], axis=1)
            pe = lax.dot_general(bo, x1e, (((0,), (0,)), ((), ())),
                                 preferred_element_type=jnp.float32)
            s1 = jnp.sum(x1, axis=0, keepdims=True)
            s2 = jnp.sum(x1 * x1, axis=0, keepdims=True)
            st = jnp.concatenate([s1, s2, jnp.zeros((6, D), jnp.float32)],
                                 axis=0)

            @pl.when(i == 0)
            def _():
                st_ref[...] = jnp.zeros_like(st_ref)
                pe_ref[...] = jnp.zeros_like(pe_ref)

            st_ref[...] += st
            pe_ref[...] += pe

        @pl.when(i == NBLK)
        def _():
            mean = st_ref[0:1, :] / float(N)
            var = st_ref[1:2, :] / float(N) - mean * mean
            sc = g_ref[...] / jnp.sqrt(var + 1e-5)
            P1 = pe_ref[:, 0:D]
            cntb = pe_ref[:, D:2 * D]
            pooled = P1 * sc + cntb * (be_ref[...] - mean * sc)
            logits = jnp.dot(pooled, lw_ref[...],
                             preferred_element_type=jnp.float32)
            o_ref[...] = jax.nn.sigmoid(logits + lb_ref[...])

    cl = lambda i: (0, jnp.minimum(i, NBLK - 1), 0)
    cl2 = lambda i: (jnp.minimum(i, NBLK - 1), 0)
    return pl.pallas_call(
        body,
        grid=(NBLK + 1,),
        in_specs=[
            pl.BlockSpec((2, RBLK, ACCW), cl),
            pl.BlockSpec((RBLK, 1), cl2),
            pl.BlockSpec((8, D), lambda i: (0, 0)),
            pl.BlockSpec((8, D), lambda i: (0, 0)),
            pl.BlockSpec((1, D), lambda i: (0, 0)),
            pl.BlockSpec((1, D), lambda i: (0, 0)),
            pl.BlockSpec((1, D), lambda i: (0, 0)),
            pl.BlockSpec((D, OUT), lambda i: (0, 0)),
            pl.BlockSpec((1, OUT), lambda i: (0, 0)),
        ],
        out_specs=[
            pl.BlockSpec((8, D), lambda i: (0, 0)),
            pl.BlockSpec((G, 2 * D), lambda i: (0, 0)),
            pl.BlockSpec((G, OUT), lambda i: (0, 0)),
        ],
        out_shape=[
            jax.ShapeDtypeStruct((8, D), jnp.float32),
            jax.ShapeDtypeStruct((G, 2 * D), jnp.float32),
            jax.ShapeDtypeStruct((G, OUT), jnp.float32),
        ],
    )(outp, batchcol, E0, E1, bias2d, gamma2d, beta2d, lin_W, lin_b2d)


def kernel(x, edge_index, batch, W, att_src, att_dst, bias_gat, gamma, beta,
           lin_W, lin_b):
    f32 = jnp.float32
    x_pad = jnp.zeros((NPAD, D), f32).at[:N].set(x)

    eye = jnp.eye(H, dtype=f32)
    Asrc = (eye[:, None, :] * att_src[:, :, None]).reshape(D, H)
    Adst = (eye[:, None, :] * att_dst[:, :, None]).reshape(D, H)
    Amat = jnp.concatenate([Asrc, Adst], axis=1)

    h2, aTr = _tc_front(x_pad, W, Amat)

    loop = jnp.arange(N, dtype=jnp.int32)
    npad_e = ETOT_PAD - (E + N)
    src = jnp.concatenate([edge_index[0].astype(jnp.int32), loop,
                           jnp.full((npad_e,), NPAD - 1, jnp.int32)])
    dst = jnp.concatenate([edge_index[1].astype(jnp.int32), loop,
                           jnp.zeros((npad_e,), jnp.int32)])
    srcdst = src | (dst << 16)

    outp = _sc_edges(aTr.reshape(2, 4 * NPAD), srcdst, h2)

    batchcol = jnp.full((NPAD, 1), G, jnp.int32).at[:N, 0].set(
        batch.astype(jnp.int32))
    hot = (jnp.eye(H, dtype=f32)[:, :, None] * jnp.ones((1, 1, C), f32)).reshape(H, D)
    E0 = jnp.concatenate([hot[0:2], jnp.zeros((6, D), f32)], axis=0)
    E1 = jnp.concatenate([hot[2:4], jnp.zeros((6, D), f32)], axis=0)

    stats, pe, out = _tc_epilogue(outp, batchcol, E0, E1,
                                  bias_gat.reshape(1, D), gamma.reshape(1, D),
                                  beta.reshape(1, D), lin_W,
                                  lin_b.reshape(1, OUT))
    del stats, pe
    return out
